# baseline (speedup 1.0000x reference)
import functools

import jax
import jax.numpy as jnp
from jax import lax
from jax.experimental import pallas as pl
from jax.experimental.pallas import tpu as pltpu
from jax.experimental.pallas import tpu_sc as plsc

_K = 32


def _lane_perm(v, idx):
    return lax.gather(
        v,
        idx[:, None],
        lax.GatherDimensionNumbers(
            offset_dims=(), collapsed_slice_dims=(0,), start_index_map=(0,)
        ),
        (1,),
        mode=lax.GatherScatterMode.PROMISE_IN_BOUNDS,
    )


@functools.lru_cache(maxsize=None)
def _build_sc_means(b: int, c: int):
    info = plsc.get_sparse_core_info()
    nc, ns, nl = info.num_cores, info.num_subcores, info.num_lanes
    nw = nc * ns
    planes = b * c
    assert planes % nw == 0 and _K == 2 * nl
    ppw = planes // nw
    assert ppw == nl and c % ppw == 0

    mesh = plsc.VectorSubcoreMesh(core_axis_name="c", subcore_axis_name="s")

    @functools.partial(
        pl.kernel,
        out_type=jax.ShapeDtypeStruct((planes,), jnp.float32),
        mesh=mesh,
        scratch_types=[
            pltpu.VMEM((ppw, _K), jnp.float32),
            pltpu.VMEM((nl,), jnp.float32),
        ],
    )
    def sc_means(in_hbm, out_hbm, head_v, mean_v):
        wid = lax.axis_index("s") * nc + lax.axis_index("c")
        bi = (wid * ppw) // c
        c0 = (wid * ppw) % c
        pltpu.sync_copy(in_hbm.at[bi, pl.ds(c0, ppw)], head_v)
        lanes = lax.iota(jnp.int32, nl)
        packed = jnp.zeros((nl,), jnp.float32)
        for r in range(ppw):
            v = head_v[r, pl.ds(0, nl)] + head_v[r, pl.ds(nl, nl)]
            sh = nl // 2
            while sh >= 1:
                v = v + _lane_perm(v, lanes ^ sh)
                sh //= 2
            packed = jnp.where(lanes == r, v * (1.0 / _K), packed)
        mean_v[...] = packed
        pltpu.sync_copy(mean_v, out_hbm.at[pl.ds(wid * ppw, ppw)])

    return sc_means


@functools.lru_cache(maxsize=None)
def _build_tc_splat(b: int, c: int, h: int, w: int):
    cb = 8

    def tc_splat(means_ref, out_ref):
        i = pl.program_id(0)
        for j in range(cb):
            out_ref[0, j] = jnp.full((h, w), means_ref[i * cb + j], jnp.float32)

    nsteps = (b * c) // cb
    return pl.pallas_call(
        tc_splat,
        grid=(nsteps,),
        in_specs=[pl.BlockSpec(memory_space=pltpu.SMEM)],
        out_specs=pl.BlockSpec(
            (1, cb, h, w), lambda i: (i // (c // cb), i % (c // cb), 0, 0)
        ),
        out_shape=jax.ShapeDtypeStruct((b, c, h, w), jnp.float32),
    )


def kernel(feat_pos):
    b, c, h, w = feat_pos.shape
    head = lax.slice(feat_pos, (0, 0, 0, 0), (b, c, 1, _K)).reshape(b, c, _K)
    means = _build_sc_means(b, c)(head)
    return _build_tc_splat(b, c, h, w)(means)


# speedup vs baseline: 1.1364x; 1.1364x over previous
import functools

import jax
import jax.numpy as jnp
from jax import lax
from jax.experimental import pallas as pl
from jax.experimental.pallas import tpu as pltpu
from jax.experimental.pallas import tpu_sc as plsc

_K = 32


def _lane_perm(v, idx):
    return lax.gather(
        v,
        idx[:, None],
        lax.GatherDimensionNumbers(
            offset_dims=(), collapsed_slice_dims=(0,), start_index_map=(0,)
        ),
        (1,),
        mode=lax.GatherScatterMode.PROMISE_IN_BOUNDS,
    )


@functools.lru_cache(maxsize=None)
def _build_sc_means(b: int, c: int):
    info = plsc.get_sparse_core_info()
    nc, ns, nl = info.num_cores, info.num_subcores, info.num_lanes
    nw = nc * ns
    planes = b * c
    assert planes % nw == 0 and _K == 2 * nl
    ppw = planes // nw
    assert ppw == nl and c % ppw == 0

    mesh = plsc.VectorSubcoreMesh(core_axis_name="c", subcore_axis_name="s")

    @functools.partial(
        pl.kernel,
        out_type=jax.ShapeDtypeStruct((planes,), jnp.float32),
        mesh=mesh,
        scratch_types=[
            pltpu.VMEM((ppw, _K), jnp.float32),
            pltpu.VMEM((nl,), jnp.float32),
        ],
    )
    def sc_means(in_hbm, out_hbm, head_v, mean_v):
        wid = lax.axis_index("s") * nc + lax.axis_index("c")
        bi = (wid * ppw) // c
        c0 = (wid * ppw) % c
        pltpu.sync_copy(in_hbm.at[bi, pl.ds(c0, ppw)], head_v)
        lanes = lax.iota(jnp.int32, nl)
        packed = jnp.zeros((nl,), jnp.float32)
        for r in range(ppw):
            v = head_v[r, pl.ds(0, nl)] + head_v[r, pl.ds(nl, nl)]
            sh = nl // 2
            while sh >= 1:
                v = v + _lane_perm(v, lanes ^ sh)
                sh //= 2
            packed = jnp.where(lanes == r, v * (1.0 / _K), packed)
        mean_v[...] = packed
        pltpu.sync_copy(mean_v, out_hbm.at[pl.ds(wid * ppw, ppw)])

    return sc_means


@functools.lru_cache(maxsize=None)
def _build_tc_splat(b: int, c: int, h: int, w: int):
    cb = 8
    n = h * w

    def tc_splat(means_ref, out_ref):
        i = pl.program_id(0)
        for j in range(cb):
            out_ref[0, j] = jnp.full((n,), means_ref[i * cb + j], jnp.float32)

    nsteps = (b * c) // cb
    return pl.pallas_call(
        tc_splat,
        grid=(nsteps,),
        in_specs=[pl.BlockSpec(memory_space=pltpu.SMEM)],
        out_specs=pl.BlockSpec(
            (1, cb, n), lambda i: (i // (c // cb), i % (c // cb), 0)
        ),
        out_shape=jax.ShapeDtypeStruct((b, c, n), jnp.float32),
    )


def kernel(feat_pos):
    b, c, h, w = feat_pos.shape
    head = lax.slice(feat_pos, (0, 0, 0, 0), (b, c, 1, _K)).reshape(b, c, _K)
    means = _build_sc_means(b, c)(head)
    return _build_tc_splat(b, c, h, w)(means).reshape(b, c, h, w)


# speedup vs baseline: 1.7064x; 1.5016x over previous
import functools

import jax
import jax.numpy as jnp
from jax import lax
from jax.experimental import pallas as pl
from jax.experimental.pallas import tpu as pltpu
from jax.experimental.pallas import tpu_sc as plsc

_K = 32


@functools.lru_cache(maxsize=None)
def _build_sc_kernel(b: int, h: int, w: int, c: int):
    info = plsc.get_sparse_core_info()
    nc, ns, nl = info.num_cores, info.num_subcores, info.num_lanes
    nw = nc * ns
    assert c % nl == 0 and nw % b == 0 and h % (nw // b) == 0
    spb = nw // b
    hb = h // spb
    groups = c // nl

    mesh = plsc.VectorSubcoreMesh(core_axis_name="c", subcore_axis_name="s")

    @functools.partial(
        pl.kernel,
        out_type=jax.ShapeDtypeStruct((b, h, w, c), jnp.float32),
        mesh=mesh,
        scratch_types=[
            pltpu.VMEM((_K, c), jnp.float32),
            pltpu.VMEM((1, w, c), jnp.float32),
            pltpu.SemaphoreType.DMA,
        ],
    )
    def sc_kernel(in_hbm, out_hbm, head_v, slab_v, sem):
        wid = lax.axis_index("s") * nc + lax.axis_index("c")
        bi = wid // spb
        h0 = (wid % spb) * hb
        pltpu.sync_copy(in_hbm.at[bi], head_v)
        mvecs = []
        for g in range(groups):
            v = head_v[0, pl.ds(g * nl, nl)]
            for k in range(1, _K):
                v = v + head_v[k, pl.ds(g * nl, nl)]
            mvecs.append(v * (1.0 / _K))

        def fill(i, carry):
            for g in range(groups):
                slab_v[0, i, pl.ds(g * nl, nl)] = mvecs[g]
            return carry

        lax.fori_loop(0, w, fill, 0)
        copies = [
            pltpu.async_copy(slab_v, out_hbm.at[bi, pl.ds(h0 + q, 1)], sem)
            for q in range(hb)
        ]
        for cp in copies:
            cp.wait()

    return sc_kernel


def kernel(feat_pos):
    b, c, h, w = feat_pos.shape
    xt = jnp.transpose(feat_pos, (0, 2, 3, 1))
    head = lax.slice(xt, (0, 0, 0, 0), (b, 1, _K, c)).reshape(b, _K, c)
    out_t = _build_sc_kernel(b, h, w, c)(head)
    return jnp.transpose(out_t, (0, 3, 1, 2))


# speedup vs baseline: 2.4532x; 1.4377x over previous
import functools

import jax
import jax.numpy as jnp
from jax import lax
from jax.experimental import pallas as pl
from jax.experimental.pallas import tpu as pltpu
from jax.experimental.pallas import tpu_sc as plsc

_K = 32


@functools.lru_cache(maxsize=None)
def _build_sc_kernel(b: int, h: int, w: int, c: int):
    info = plsc.get_sparse_core_info()
    nc, ns, nl = info.num_cores, info.num_subcores, info.num_lanes
    nw = nc * ns
    assert c % nl == 0 and nw % b == 0 and h % (nw // b) == 0
    spb = nw // b
    hb = h // spb
    groups = c // nl

    mesh = plsc.VectorSubcoreMesh(core_axis_name="c", subcore_axis_name="s")

    @functools.partial(
        pl.kernel,
        out_type=jax.ShapeDtypeStruct((b, h, w, c), jnp.float32),
        mesh=mesh,
        scratch_types=[
            pltpu.VMEM((_K, c), jnp.float32),
            pltpu.VMEM((1, w, c), jnp.float32),
            pltpu.SemaphoreType.DMA,
        ],
    )
    def sc_kernel(in_hbm, out_hbm, head_v, slab_v, sem):
        wid = lax.axis_index("s") * nc + lax.axis_index("c")
        bi = wid // spb
        h0 = (wid % spb) * hb
        pltpu.sync_copy(in_hbm.at[bi], head_v)
        mvecs = []
        for g in range(groups):
            v = head_v[0, pl.ds(g * nl, nl)]
            for k in range(1, _K):
                v = v + head_v[k, pl.ds(g * nl, nl)]
            mvecs.append(v * (1.0 / _K))

        def fill(i, carry):
            for g in range(groups):
                slab_v[0, i, pl.ds(g * nl, nl)] = mvecs[g]
            return carry

        lax.fori_loop(0, w, fill, 0)
        copies = [
            pltpu.async_copy(slab_v, out_hbm.at[bi, pl.ds(h0 + q, 1)], sem)
            for q in range(hb)
        ]
        for cp in copies:
            cp.wait()

    return sc_kernel


def kernel(feat_pos):
    b, c, h, w = feat_pos.shape
    head = jnp.transpose(
        lax.slice(feat_pos, (0, 0, 0, 0), (b, c, 1, _K)).reshape(b, c, _K),
        (0, 2, 1),
    )
    out_t = _build_sc_kernel(b, h, w, c)(head)
    return jnp.transpose(out_t, (0, 3, 1, 2))
